# baseline (speedup 1.0000x reference)
import functools

import jax
import jax.numpy as jnp
from jax import lax
from jax.experimental import pallas as pl
from jax.experimental.pallas import tpu as pltpu

_N = 2048
_D = 128
_K = 384
_PIT = 24


def _power_kernel(t_ref, v_ref, rho_ref):
    def body(_, v):
        w = jnp.dot(t_ref[...], v, preferred_element_type=jnp.float32)
        nrm = jnp.sqrt(jnp.sum(w * w, axis=0, keepdims=True))
        return w / jnp.maximum(nrm, 1e-30)
    v = body(0, v_ref[...])
    v = lax.fori_loop(0, _PIT - 1, body, v)
    w = jnp.dot(t_ref[...], v, preferred_element_type=jnp.float32)
    nrm = jnp.sqrt(jnp.sum(w * w, axis=0))
    rho_ref[0, 0] = jnp.max(nrm)


def _estimate_rho(ts):
    n = ts.shape[0]
    i = jnp.arange(n, dtype=jnp.float32)
    cols = [jnp.ones((n,), jnp.float32)]
    for p in (1.0, 2.0, 3.0, 5.0, 7.0, 11.0, 13.0):
        cols.append(jnp.sin(0.7318 * p * i + 0.25 * p))
    v0 = jnp.stack(cols, axis=1)
    v0 = v0 / jnp.sqrt(jnp.sum(v0 * v0, axis=0, keepdims=True))
    rho = pl.pallas_call(
        _power_kernel,
        out_shape=jax.ShapeDtypeStruct((1, 1), jnp.float32),
        in_specs=[
            pl.BlockSpec(memory_space=pltpu.VMEM),
            pl.BlockSpec(memory_space=pltpu.VMEM),
        ],
        out_specs=pl.BlockSpec(memory_space=pltpu.SMEM),
    )(ts, v0)
    return rho[0, 0]


def _mv3(th_ref, tl_ref, b):
    bh = b.astype(jnp.bfloat16)
    bl = (b - bh.astype(jnp.float32)).astype(jnp.bfloat16)
    th = th_ref[...]
    return (jnp.dot(th, bh, preferred_element_type=jnp.float32)
            + jnp.dot(th, bl, preferred_element_type=jnp.float32)
            + jnp.dot(tl_ref[...], bh, preferred_element_type=jnp.float32))


def _cheb_recurrence(th_ref, tl_ref, v, c_ref, t0, t1, emit):
    t0[...] = v
    t1[...] = _mv3(th_ref, tl_ref, v)
    for i in range(4):
        emit(i, c_ref[0, i] * t0[...] + c_ref[1, i] * t1[...], True)

    def body(j, _):
        t0[...] = 2.0 * _mv3(th_ref, tl_ref, t1[...]) - t0[...]
        for i in range(4):
            emit(i, c_ref[2 * j, i] * t0[...], False)
        t1[...] = 2.0 * _mv3(th_ref, tl_ref, t0[...]) - t1[...]
        for i in range(4):
            emit(i, c_ref[2 * j + 1, i] * t1[...], False)
        return 0

    lax.fori_loop(1, (_K + 2) // 2, body, 0)


def _stage1_kernel(th_ref, tl_ref, v_ref, c_ref, o_ref, t0, t1):
    w = v_ref.shape[1]

    def emit(i, val, init):
        sl = (slice(None), slice(i * w, (i + 1) * w))
        if init:
            o_ref[sl] = val
        else:
            o_ref[sl] += val

    _cheb_recurrence(th_ref, tl_ref, v_ref[...], c_ref, t0, t1, emit)
    for i in range(4):
        sl = (slice(None), slice(i * w, (i + 1) * w))
        o_ref[sl] = jnp.abs(o_ref[sl])


def _stage2_kernel(th_ref, tl_ref, v_ref, c_ref, o_ref, t0, t1):
    def emit(i, val, init):
        if init:
            o_ref[i] = val
        else:
            o_ref[i] += val

    _cheb_recurrence(th_ref, tl_ref, v_ref[...], c_ref, t0, t1, emit)
    for i in range(4):
        o_ref[i] = jnp.abs(o_ref[i])


def _cheb_stage1(th, tl, v, coefs):
    n, w = v.shape
    return pl.pallas_call(
        _stage1_kernel,
        grid=(1,),
        in_specs=[
            pl.BlockSpec((n, n), lambda j: (0, 0)),
            pl.BlockSpec((n, n), lambda j: (0, 0)),
            pl.BlockSpec((n, w), lambda j: (0, 0)),
            pl.BlockSpec(memory_space=pltpu.SMEM),
        ],
        out_specs=pl.BlockSpec((n, 4 * w), lambda j: (0, 0)),
        out_shape=jax.ShapeDtypeStruct((n, 4 * w), jnp.float32),
        scratch_shapes=[
            pltpu.VMEM((n, w), jnp.float32),
            pltpu.VMEM((n, w), jnp.float32),
        ],
    )(th, tl, v, coefs)


_W2 = 128


def _cheb_stage2(th, tl, v, coefs):
    n, w = v.shape
    nblk = w // _W2
    return pl.pallas_call(
        _stage2_kernel,
        grid=(nblk,),
        in_specs=[
            pl.BlockSpec((n, n), lambda j: (0, 0)),
            pl.BlockSpec((n, n), lambda j: (0, 0)),
            pl.BlockSpec((n, _W2), lambda j: (0, j)),
            pl.BlockSpec(memory_space=pltpu.SMEM),
        ],
        out_specs=pl.BlockSpec((4, n, _W2), lambda j: (0, 0, j)),
        out_shape=jax.ShapeDtypeStruct((4, n, w), jnp.float32),
        scratch_shapes=[
            pltpu.VMEM((n, _W2), jnp.float32),
            pltpu.VMEM((n, _W2), jnp.float32),
        ],
    )(th, tl, v, coefs)


def kernel(x, edge_index):
    n = x.shape[0]
    d = x.shape[1]
    row = edge_index[0]
    col = edge_index[1]
    ones = jnp.ones((edge_index.shape[1],), dtype=x.dtype)
    deg = jnp.zeros((n,), dtype=x.dtype).at[col].add(ones)
    deg_half = deg ** (-0.5)
    deg_half = jnp.where(jnp.isinf(deg_half), 0.0, deg_half)
    w = deg_half[row] * deg_half[col]
    t = jnp.zeros((n, n), dtype=x.dtype).at[row, col].add(w)
    ts = 0.5 * (t + t.T)

    gersh = jnp.max(jnp.sum(jnp.abs(ts), axis=1))
    rho = _estimate_rho(ts)
    dom = jnp.minimum(gersh, jnp.maximum(rho * 1.06, 1.12))

    k = jnp.arange(_K + 1, dtype=jnp.float32)
    xs = jnp.cos(jnp.pi * (k + 0.5) / (_K + 1))
    ls = dom * xs
    l2 = ls * ls
    l4 = l2 * l2
    l8 = l4 * l4
    l16 = l8 * l8
    gvals = jnp.stack([
        jnp.sqrt(jnp.clip(ls - l2, 0.0, None)),
        jnp.sqrt(jnp.clip(l2 - l4, 0.0, None)),
        jnp.sqrt(jnp.clip(l4 - l8, 0.0, None)),
        jnp.sqrt(jnp.clip(l8 - l16, 0.0, None)),
    ], axis=0)
    j = jnp.arange(_K + 1, dtype=jnp.float32)
    cosm = jnp.cos(jnp.pi * j[:, None] * (k[None, :] + 0.5) / (_K + 1))
    coefs = (2.0 / (_K + 1)) * (gvals @ cosm.T)
    coefs = coefs.at[:, 0].mul(0.5)
    coefs = jnp.pad(coefs, ((0, 0), (0, 1)))
    coefs_t = coefs.T

    ts_scaled = ts / dom
    th = ts_scaled.astype(jnp.bfloat16)
    tl = (ts_scaled - th.astype(jnp.float32)).astype(jnp.bfloat16)

    s1_mine = _cheb_stage1(th, tl, x, coefs_t)
    s2_3 = _cheb_stage2(th, tl, s1_mine, coefs_t)

    s1_ref = s1_mine.reshape(n, 4, d).transpose(0, 2, 1).reshape(n, 4 * d)
    s2_ref = (s2_3.reshape(4, n, 4, d)
              .transpose(1, 3, 2, 0).reshape(n, 16 * d))
    return jnp.concatenate([x, s1_ref, s2_ref], axis=1)


# speedup vs baseline: 2.8596x; 2.8596x over previous
import functools

import jax
import jax.numpy as jnp
from jax import lax
from jax.experimental import pallas as pl
from jax.experimental.pallas import tpu as pltpu

_N = 2048
_D = 128
_K = 384
_PIT = 24


def _power_kernel(t_ref, v_ref, rho_ref):
    def body(_, v):
        w = jnp.dot(t_ref[...], v, preferred_element_type=jnp.float32)
        nrm = jnp.sqrt(jnp.sum(w * w, axis=0, keepdims=True))
        return w / jnp.maximum(nrm, 1e-30)
    v = body(0, v_ref[...])
    v = lax.fori_loop(0, _PIT - 1, body, v)
    w = jnp.dot(t_ref[...], v, preferred_element_type=jnp.float32)
    nrm = jnp.sqrt(jnp.sum(w * w, axis=0))
    rho_ref[0, 0] = jnp.max(nrm)


def _estimate_rho(ts):
    n = ts.shape[0]
    i = jnp.arange(n, dtype=jnp.float32)
    cols = [jnp.ones((n,), jnp.float32)]
    for p in (1.0, 2.0, 3.0, 5.0, 7.0, 11.0, 13.0):
        cols.append(jnp.sin(0.7318 * p * i + 0.25 * p))
    v0 = jnp.stack(cols, axis=1)
    v0 = v0 / jnp.sqrt(jnp.sum(v0 * v0, axis=0, keepdims=True))
    rho = pl.pallas_call(
        _power_kernel,
        out_shape=jax.ShapeDtypeStruct((1, 1), jnp.float32),
        in_specs=[
            pl.BlockSpec(memory_space=pltpu.VMEM),
            pl.BlockSpec(memory_space=pltpu.VMEM),
        ],
        out_specs=pl.BlockSpec(memory_space=pltpu.SMEM),
    )(ts, v0)
    return rho[0, 0]


def _mv(t_ref, b):
    return jnp.dot(t_ref[...], b, preferred_element_type=jnp.float32)


def _cheb_recurrence(t_ref, v, c_ref, t0, t1, emit):
    t0[...] = v
    t1[...] = _mv(t_ref, v)
    for i in range(4):
        emit(i, c_ref[0, i] * t0[...] + c_ref[1, i] * t1[...], True)

    def body(j, _):
        t0[...] = 2.0 * _mv(t_ref, t1[...]) - t0[...]
        for i in range(4):
            emit(i, c_ref[2 * j, i] * t0[...], False)
        t1[...] = 2.0 * _mv(t_ref, t0[...]) - t1[...]
        for i in range(4):
            emit(i, c_ref[2 * j + 1, i] * t1[...], False)
        return 0

    lax.fori_loop(1, (_K + 2) // 2, body, 0)


def _stage1_kernel(t_ref, v_ref, c_ref, o_ref, t0, t1):
    w = v_ref.shape[1]

    def emit(i, val, init):
        sl = (slice(None), slice(i * w, (i + 1) * w))
        if init:
            o_ref[sl] = val
        else:
            o_ref[sl] += val

    _cheb_recurrence(t_ref, v_ref[...], c_ref, t0, t1, emit)
    for i in range(4):
        sl = (slice(None), slice(i * w, (i + 1) * w))
        o_ref[sl] = jnp.abs(o_ref[sl])


def _stage2_kernel(t_ref, v_ref, c_ref, o_ref, t0, t1):
    def emit(i, val, init):
        if init:
            o_ref[i] = val
        else:
            o_ref[i] += val

    _cheb_recurrence(t_ref, v_ref[...], c_ref, t0, t1, emit)
    for i in range(4):
        o_ref[i] = jnp.abs(o_ref[i])


def _cheb_stage1(ts, v, coefs):
    n, w = v.shape
    return pl.pallas_call(
        _stage1_kernel,
        grid=(1,),
        in_specs=[
            pl.BlockSpec((n, n), lambda j: (0, 0)),
            pl.BlockSpec((n, w), lambda j: (0, 0)),
            pl.BlockSpec(memory_space=pltpu.SMEM),
        ],
        out_specs=pl.BlockSpec((n, 4 * w), lambda j: (0, 0)),
        out_shape=jax.ShapeDtypeStruct((n, 4 * w), jnp.float32),
        scratch_shapes=[
            pltpu.VMEM((n, w), jnp.float32),
            pltpu.VMEM((n, w), jnp.float32),
        ],
    )(ts, v, coefs)


_W2 = 128


def _cheb_stage2(ts, v, coefs):
    n, w = v.shape
    nblk = w // _W2
    return pl.pallas_call(
        _stage2_kernel,
        grid=(nblk,),
        in_specs=[
            pl.BlockSpec((n, n), lambda j: (0, 0)),
            pl.BlockSpec((n, _W2), lambda j: (0, j)),
            pl.BlockSpec(memory_space=pltpu.SMEM),
        ],
        out_specs=pl.BlockSpec((4, n, _W2), lambda j: (0, 0, j)),
        out_shape=jax.ShapeDtypeStruct((4, n, w), jnp.float32),
        scratch_shapes=[
            pltpu.VMEM((n, _W2), jnp.float32),
            pltpu.VMEM((n, _W2), jnp.float32),
        ],
    )(ts, v, coefs)


def kernel(x, edge_index):
    n = x.shape[0]
    d = x.shape[1]
    row = edge_index[0]
    col = edge_index[1]
    ones = jnp.ones((edge_index.shape[1],), dtype=x.dtype)
    deg = jnp.zeros((n,), dtype=x.dtype).at[col].add(ones)
    deg_half = deg ** (-0.5)
    deg_half = jnp.where(jnp.isinf(deg_half), 0.0, deg_half)
    w = deg_half[row] * deg_half[col]
    t = jnp.zeros((n, n), dtype=x.dtype).at[row, col].add(w)
    ts = 0.5 * (t + t.T)

    gersh = jnp.max(jnp.sum(jnp.abs(ts), axis=1))
    rho = _estimate_rho(ts)
    dom = jnp.minimum(gersh, jnp.maximum(rho * 1.06, 1.12))

    k = jnp.arange(_K + 1, dtype=jnp.float32)
    xs = jnp.cos(jnp.pi * (k + 0.5) / (_K + 1))
    ls = dom * xs
    l2 = ls * ls
    l4 = l2 * l2
    l8 = l4 * l4
    l16 = l8 * l8
    gvals = jnp.stack([
        jnp.sqrt(jnp.clip(ls - l2, 0.0, None)),
        jnp.sqrt(jnp.clip(l2 - l4, 0.0, None)),
        jnp.sqrt(jnp.clip(l4 - l8, 0.0, None)),
        jnp.sqrt(jnp.clip(l8 - l16, 0.0, None)),
    ], axis=0)
    j = jnp.arange(_K + 1, dtype=jnp.float32)
    cosm = jnp.cos(jnp.pi * j[:, None] * (k[None, :] + 0.5) / (_K + 1))
    coefs = (2.0 / (_K + 1)) * (gvals @ cosm.T)
    coefs = coefs.at[:, 0].mul(0.5)
    coefs = jnp.pad(coefs, ((0, 0), (0, 1)))
    coefs_t = coefs.T

    ts_scaled = ts / dom

    s1_mine = _cheb_stage1(ts_scaled, x, coefs_t)
    s2_3 = _cheb_stage2(ts_scaled, s1_mine, coefs_t)

    s1_ref = s1_mine.reshape(n, 4, d).transpose(0, 2, 1).reshape(n, 4 * d)
    s2_ref = (s2_3.reshape(4, n, 4, d)
              .transpose(1, 3, 2, 0).reshape(n, 16 * d))
    return jnp.concatenate([x, s1_ref, s2_ref], axis=1)
